# baseline (speedup 1.0000x reference)
import functools

import jax
import jax.numpy as jnp
from jax import lax
from jax.experimental import pallas as pl
from jax.experimental.pallas import tpu as pltpu
from jax.experimental.pallas import tpu_sc as plsc

B = 64
C = 32
H4 = 64
W4 = 192
HW = H4 * W4
CUT = HW // 2
L = 16
NBLK = HW // L
NC = 2
NS = 16
NW = NC * NS
HB = B // 2
U = 8
PW = 4 * L
INF_BITS = 0x7F800000
NBIS = 16
RB = 8


def _abssum_body(x_ref, o_ref, mm_ref):
    a = jnp.sum(jnp.abs(x_ref[...]), axis=1)
    o_ref[...] = a.reshape(RB, HW)
    mn = jnp.min(a, axis=(1, 2))
    mx = jnp.max(a, axis=(1, 2))
    mm_ref[...] = jnp.concatenate(
        [jnp.broadcast_to(mn[:, None], (RB, L)),
         jnp.broadcast_to(mx[:, None], (RB, L))], axis=1)


def _abssum_half(x, half):
    off = half * (HB // RB)
    return pl.pallas_call(
        _abssum_body,
        grid=(HB // RB,),
        in_specs=[pl.BlockSpec((RB, C, H4, W4),
                               lambda i, off=off: (i + off, 0, 0, 0))],
        out_specs=[pl.BlockSpec((RB, HW), lambda i: (i, 0)),
                   pl.BlockSpec((RB, 2 * L), lambda i: (i, 0))],
        out_shape=[jax.ShapeDtypeStruct((HB, HW), jnp.float32),
                   jax.ShapeDtypeStruct((HB, 2 * L), jnp.float32)],
    )(x)


def _select_body(s_hbm, mm_hbm, out_hbm, sbuf, mbuf, ovec, sem, msem):
    wid = lax.axis_index("s") * NC + lax.axis_index("c")

    cp = pltpu.make_async_copy(s_hbm.at[wid], sbuf, sem)
    cp.start()
    cpm = pltpu.make_async_copy(mm_hbm.at[wid], mbuf, msem)
    cpm.start()

    def bis_pass(s, state):
        lo, hi = state
        w = hi - lo
        m1 = lo + w // 4
        m2 = lo + w // 2
        m3 = lo + (w // 2 + w // 4)
        f1 = lax.bitcast_convert_type(m1, jnp.float32)
        f2 = lax.bitcast_convert_type(m2, jnp.float32)
        f3 = lax.bitcast_convert_type(m3, jnp.float32)

        def cbody(i, accs):
            a1, a2, a3 = accs
            for u in range(U):
                off = (i * U + u) * L
                f = s[pl.ds(off, L)]
                a1 += jnp.where(f <= f1, 1, 0)
                a2 += jnp.where(f <= f2, 1, 0)
                a3 += jnp.where(f <= f3, 1, 0)
            return (a1, a2, a3)

        z = jnp.zeros((L,), jnp.int32)
        a1, a2, a3 = lax.fori_loop(0, NBLK // U, cbody, (z, z, z))
        c1, c2, c3 = a1[0], a2[0], a3[0]
        for j in range(1, L):
            c1 = c1 + a1[j]
            c2 = c2 + a2[j]
            c3 = c3 + a3[j]
        in1 = c1 >= CUT
        in2 = c2 >= CUT
        in3 = c3 >= CUT
        new_lo = jnp.where(in1, lo,
                           jnp.where(in2, m1 + 1,
                                     jnp.where(in3, m2 + 1, m3 + 1)))
        new_hi = jnp.where(in1, m1,
                           jnp.where(in2, m2,
                                     jnp.where(in3, m3, hi)))
        return (new_lo, new_hi)

    def emit_row(s, t_bits, b):
        t_val = lax.bitcast_convert_type(t_bits, jnp.float32)

        def sum_body(i, carry):
            sv, qv, cv = carry
            for u in range(U):
                off = (i * U + u) * L
                f = s[pl.ds(off, L)]
                m = f < t_val
                fm = jnp.where(m, f, 0.0)
                sv += fm
                qv += fm * fm
                cv += jnp.where(m, 1, 0)
            return (sv, qv, cv)

        sv, qv, cv = lax.fori_loop(
            0, NBLK // U, sum_body,
            (jnp.zeros((L,), jnp.float32), jnp.zeros((L,), jnp.float32),
             jnp.zeros((L,), jnp.int32)))

        ovec[pl.ds(0, L)] = sv
        ovec[pl.ds(L, L)] = qv
        ovec[pl.ds(2 * L, L)] = cv.astype(jnp.float32)
        ovec[pl.ds(3 * L, L)] = jnp.full((L,), t_val, jnp.float32)
        pltpu.sync_copy(ovec, out_hbm.at[b])

    cp.wait()
    cpm.wait()
    mn_v = mbuf[pl.ds(0, L)]
    mx_v = mbuf[pl.ds(L, L)]
    lo0 = lax.bitcast_convert_type(mn_v[0], jnp.int32)
    hi0 = lax.bitcast_convert_type(mx_v[0], jnp.int32)

    def bis_body(_, state):
        return lax.cond(state[0] < state[1],
                        lambda st: bis_pass(sbuf, st),
                        lambda st: st,
                        state)

    state = lax.fori_loop(0, NBIS, bis_body, (lo0, hi0))
    emit_row(sbuf, state[0], wid)


_select = functools.partial(
    pl.kernel,
    out_type=jax.ShapeDtypeStruct((HB, PW), jnp.float32),
    mesh=plsc.VectorSubcoreMesh(core_axis_name="c", subcore_axis_name="s"),
    scratch_types=[
        pltpu.VMEM((HW,), jnp.float32),
        pltpu.VMEM((2 * L,), jnp.float32),
        pltpu.VMEM((PW,), jnp.float32),
        pltpu.SemaphoreType.DMA,
        pltpu.SemaphoreType.DMA,
    ],
)(_select_body)


def _combine_body(pa_ref, pb_ref, o_ref):
    p = jnp.concatenate([pa_ref[...], pb_ref[...]], axis=0)
    sum_lt = jnp.sum(p[:, 0:L], axis=1, keepdims=True)
    sumsq_lt = jnp.sum(p[:, L:2 * L], axis=1, keepdims=True)
    cnt_lt = p[:, 2 * L:2 * L + 1]
    t = p[:, 3 * L:3 * L + 1]
    n_tie = CUT - cnt_lt
    sum_b = sum_lt + n_tie * t
    sumsq_b = sumsq_lt + n_tie * t * t
    n_total = B * CUT
    s_tot = jnp.sum(sum_b)
    q_tot = jnp.sum(sumsq_b)
    var = (q_tot - s_tot * s_tot / n_total) / (n_total - 1)
    o_ref[...] = jnp.broadcast_to(jnp.sqrt(var), (1, 1))


def kernel(input):
    sa, mma = _abssum_half(input, 0)
    pa = _select(sa, mma)
    sb, mmb = _abssum_half(input, 1)
    pb = _select(sb, mmb)
    out = pl.pallas_call(
        _combine_body,
        out_shape=jax.ShapeDtypeStruct((1, 1), jnp.float32),
    )(pa, pb)
    return out.reshape(())


# speedup vs baseline: 1.2198x; 1.2198x over previous
import functools

import jax
import jax.numpy as jnp
from jax import lax
from jax.experimental import pallas as pl
from jax.experimental.pallas import tpu as pltpu
from jax.experimental.pallas import tpu_sc as plsc

B = 64
C = 32
H4 = 64
W4 = 192
HW = H4 * W4
CUT = HW // 2
L = 16
NBLK = HW // L
NC = 2
NS = 16
NW = NC * NS
HB = B // 2
U = 8
PW = 4 * L
INF_BITS = 0x7F800000
NBIS = 31
RB = 8


def _abssum_body(x_ref, o_ref, mm_ref):
    a = jnp.sum(jnp.abs(x_ref[...]), axis=1)
    o_ref[...] = a.reshape(RB, HW)
    mn = jnp.min(a, axis=(1, 2))
    mx = jnp.max(a, axis=(1, 2))
    mm_ref[...] = jnp.concatenate(
        [jnp.broadcast_to(mn[:, None], (RB, L)),
         jnp.broadcast_to(mx[:, None], (RB, L))], axis=1)


def _abssum_half(x, half):
    off = half * (HB // RB)
    return pl.pallas_call(
        _abssum_body,
        grid=(HB // RB,),
        in_specs=[pl.BlockSpec((RB, C, H4, W4),
                               lambda i, off=off: (i + off, 0, 0, 0))],
        out_specs=[pl.BlockSpec((RB, HW), lambda i: (i, 0)),
                   pl.BlockSpec((RB, 2 * L), lambda i: (i, 0))],
        out_shape=[jax.ShapeDtypeStruct((HB, HW), jnp.float32),
                   jax.ShapeDtypeStruct((HB, 2 * L), jnp.float32)],
    )(x)


def _select_body(s_hbm, mm_hbm, out_hbm, sbuf, mbuf, ovec, sem, msem):
    wid = lax.axis_index("s") * NC + lax.axis_index("c")

    cp = pltpu.make_async_copy(s_hbm.at[wid], sbuf, sem)
    cp.start()
    cpm = pltpu.make_async_copy(mm_hbm.at[wid], mbuf, msem)
    cpm.start()

    def bis_pass(s, state):
        lo, hi = state
        mid = lo + (hi - lo) // 2
        mid_f = lax.bitcast_convert_type(mid, jnp.float32)

        def cbody(i, acc):
            for u in range(U):
                off = (i * U + u) * L
                acc += jnp.where(s[pl.ds(off, L)] <= mid_f, 1, 0)
            return acc

        acc = lax.fori_loop(0, NBLK // U, cbody,
                            jnp.zeros((L,), jnp.int32))
        cnt = acc[0]
        for j in range(1, L):
            cnt = cnt + acc[j]
        take_lo = cnt >= CUT
        return (jnp.where(take_lo, lo, mid + 1),
                jnp.where(take_lo, mid, hi))

    def emit_row(s, t_bits, b):
        t_val = lax.bitcast_convert_type(t_bits, jnp.float32)

        def sum_body(i, carry):
            sv, qv, cv = carry
            for u in range(U):
                off = (i * U + u) * L
                f = s[pl.ds(off, L)]
                m = f < t_val
                fm = jnp.where(m, f, 0.0)
                sv += fm
                qv += fm * fm
                cv += jnp.where(m, 1, 0)
            return (sv, qv, cv)

        sv, qv, cv = lax.fori_loop(
            0, NBLK // U, sum_body,
            (jnp.zeros((L,), jnp.float32), jnp.zeros((L,), jnp.float32),
             jnp.zeros((L,), jnp.int32)))

        ovec[pl.ds(0, L)] = sv
        ovec[pl.ds(L, L)] = qv
        ovec[pl.ds(2 * L, L)] = cv.astype(jnp.float32)
        ovec[pl.ds(3 * L, L)] = jnp.full((L,), t_val, jnp.float32)
        pltpu.sync_copy(ovec, out_hbm.at[b])

    cp.wait()
    cpm.wait()
    mn_v = mbuf[pl.ds(0, L)]
    mx_v = mbuf[pl.ds(L, L)]
    lo0 = lax.bitcast_convert_type(mn_v[0], jnp.int32)
    hi0 = lax.bitcast_convert_type(mx_v[0], jnp.int32)

    def bis_body(_, state):
        return lax.cond(state[0] < state[1],
                        lambda st: bis_pass(sbuf, st),
                        lambda st: st,
                        state)

    state = lax.fori_loop(0, NBIS, bis_body, (lo0, hi0))
    emit_row(sbuf, state[0], wid)


_select = functools.partial(
    pl.kernel,
    out_type=jax.ShapeDtypeStruct((HB, PW), jnp.float32),
    mesh=plsc.VectorSubcoreMesh(core_axis_name="c", subcore_axis_name="s"),
    scratch_types=[
        pltpu.VMEM((HW,), jnp.float32),
        pltpu.VMEM((2 * L,), jnp.float32),
        pltpu.VMEM((PW,), jnp.float32),
        pltpu.SemaphoreType.DMA,
        pltpu.SemaphoreType.DMA,
    ],
)(_select_body)


def _combine_body(pa_ref, pb_ref, o_ref):
    p = jnp.concatenate([pa_ref[...], pb_ref[...]], axis=0)
    sum_lt = jnp.sum(p[:, 0:L], axis=1, keepdims=True)
    sumsq_lt = jnp.sum(p[:, L:2 * L], axis=1, keepdims=True)
    cnt_lt = p[:, 2 * L:2 * L + 1]
    t = p[:, 3 * L:3 * L + 1]
    n_tie = CUT - cnt_lt
    sum_b = sum_lt + n_tie * t
    sumsq_b = sumsq_lt + n_tie * t * t
    n_total = B * CUT
    s_tot = jnp.sum(sum_b)
    q_tot = jnp.sum(sumsq_b)
    var = (q_tot - s_tot * s_tot / n_total) / (n_total - 1)
    o_ref[...] = jnp.broadcast_to(jnp.sqrt(var), (1, 1))


def kernel(input):
    sa, mma = _abssum_half(input, 0)
    pa = _select(sa, mma)
    sb, mmb = _abssum_half(input, 1)
    pb = _select(sb, mmb)
    out = pl.pallas_call(
        _combine_body,
        out_shape=jax.ShapeDtypeStruct((1, 1), jnp.float32),
    )(pa, pb)
    return out.reshape(())
